# baseline (speedup 1.0000x reference)
import functools
import math

import jax
import jax.numpy as jnp
from jax import lax
from jax.experimental import pallas as pl
from jax.experimental.pallas import tpu as pltpu
from jax.experimental.pallas import tpu_sc as plsc

N = 4096
H = 1024
F = 2048
E = 8
BLK = 256
MAX_BLOCKS = (2 * N) // BLK + (E - 1)
MAXPAD = MAX_BLOCKS * BLK
CB = 512
RB = 512
EPS = 1e-5

NEG = -1e30

def _router_plan_kernel(x_ref, rw_ref, rb_ref, pos_ref, gates_ref, be_ref,
                        nxt_ref, sl_ref, xb_ref):
    x = x_ref[...]
    xb_ref[...] = x.astype(jnp.bfloat16)
    rw = rw_ref[...]
    rb = rb_ref[...]
    logits = lax.dot_general(rw, x, (((1,), (1,)), ((), ())),
                             preferred_element_type=jnp.float32)
    logits = logits + rb

    r8 = lax.broadcasted_iota(jnp.int32, (E, E), 0)
    c8 = lax.broadcasted_iota(jnp.int32, (E, E), 1)
    lt8_incl = (r8 >= c8).astype(jnp.float32)
    lt8_excl = (r8 > c8).astype(jnp.float32)

    def first_argmax_onehot(lg):
        m = jnp.max(lg, axis=0, keepdims=True)
        eq = (lg == m).astype(jnp.float32)
        csum = lax.dot_general(lt8_incl, eq, (((1,), (0,)), ((), ())),
                               preferred_element_type=jnp.float32)
        oh = eq * (csum == 1.0).astype(jnp.float32)
        return m, oh

    m1, oh0 = first_argmax_onehot(logits)
    logits2 = jnp.where(oh0 > 0.0, NEG, logits)
    m2, oh1 = first_argmax_onehot(logits2)

    g0 = 1.0 / (1.0 + jnp.exp(m2 - m1))
    g1 = 1.0 - g0

    rc = lax.broadcasted_iota(jnp.int32, (CB, CB), 0)
    cc = lax.broadcasted_iota(jnp.int32, (CB, CB), 1)
    ut = (rc < cc).astype(jnp.float32)

    def ranks(oh):
        parts = []
        run = jnp.zeros((E, 1), jnp.float32)
        for cb in range(N // CB):
            ohb = oh[:, cb * CB:(cb + 1) * CB]
            loc = lax.dot_general(ohb, ut, (((1,), (0,)), ((), ())),
                                  preferred_element_type=jnp.float32)
            rank = loc + run
            parts.append(jnp.sum(rank * ohb, axis=0, keepdims=True))
            run = run + jnp.sum(ohb, axis=1, keepdims=True)
        return jnp.concatenate(parts, axis=1), run

    r0, cnt0 = ranks(oh0)
    r1, cnt1 = ranks(oh1)

    counts = cnt0 + cnt1
    nblk = jnp.floor((counts + (BLK - 1)) / BLK)
    blk_start = lax.dot_general(lt8_excl, nblk, (((1,), (0,)), ((), ())),
                                preferred_element_type=jnp.float32)
    padded_off = blk_start * BLK

    pos0 = r0 + jnp.sum(padded_off * oh0, axis=0, keepdims=True)
    pos1 = r1 + jnp.sum((padded_off + cnt0) * oh1, axis=0, keepdims=True)
    pos_ref[...] = jnp.concatenate([pos0, pos1], axis=0).astype(jnp.int32)
    gates_ref[...] = jnp.concatenate([g0, g1], axis=0)

    cum_end = blk_start + nblk
    nonempty = nblk > 0.0
    e_col = lax.broadcasted_iota(jnp.int32, (E, 1), 0).astype(jnp.float32)
    last_e = jnp.max(jnp.where(nonempty, e_col, 0.0), axis=0, keepdims=True)

    b_iota = lax.broadcasted_iota(jnp.int32, (8, 128), 1).astype(jnp.float32)
    be = jnp.zeros((8, 128), jnp.float32)
    sl = jnp.zeros((8, 128), jnp.float32)
    for e in range(E):
        past_end = (b_iota >= cum_end[e:e + 1, 0:1]).astype(jnp.float32)
        be = be + past_end
        sl = sl + jnp.where(nonempty[e:e + 1, 0:1], past_end, 0.0)
    be = jnp.minimum(be, last_e)
    sl = sl - 2.0 * jnp.floor(sl * 0.5)

    nxt_for = [None] * E
    nxt_for[E - 1] = jnp.zeros((1, 1), jnp.float32) + float(E - 1)
    for e in range(E - 2, -1, -1):
        cand = nxt_for[e + 1]
        cand = jnp.where(cand == float(e + 1), float(e), cand)
        nxt_for[e] = jnp.where(nonempty[e + 1:e + 2, 0:1], float(e + 1), cand)
    nxt = jnp.zeros((8, 128), jnp.float32)
    for e in range(E):
        in_region = ((b_iota >= blk_start[e:e + 1, 0:1])
                     & (b_iota < cum_end[e:e + 1, 0:1])).astype(jnp.float32)
        nxt = nxt + nxt_for[e] * in_region

    be_ref[...] = be.astype(jnp.int32)
    nxt_ref[...] = nxt.astype(jnp.int32)
    sl_ref[...] = sl.astype(jnp.int32)


def _expert_ffn_kernel(be_ref, nxt_ref, sl_ref, x_ref, w1_hbm, b1_ref,
                       w2_hbm, b2_ref, y_ref, w1b, w2b, s1, s2):
    i = pl.program_id(0)
    e = be_ref[i]
    sl = sl_ref[i]
    nxt = nxt_ref[i]
    prev_e = be_ref[jnp.maximum(i - 1, 0)]
    boundary = jnp.logical_or(i == 0, e != prev_e)

    @pl.when(i == 0)
    def _():
        pltpu.make_async_copy(w1_hbm.at[e], w1b.at[sl], s1.at[sl]).start()
        pltpu.make_async_copy(w2_hbm.at[e], w2b.at[sl], s2.at[sl]).start()

    @pl.when(jnp.logical_and(boundary, nxt != e))
    def _():
        nsl = 1 - sl
        pltpu.make_async_copy(w1_hbm.at[nxt], w1b.at[nsl], s1.at[nsl]).start()
        pltpu.make_async_copy(w2_hbm.at[nxt], w2b.at[nsl], s2.at[nsl]).start()

    @pl.when(boundary)
    def _():
        pltpu.make_async_copy(w1_hbm.at[e], w1b.at[sl], s1.at[sl]).wait()
        pltpu.make_async_copy(w2_hbm.at[e], w2b.at[sl], s2.at[sl]).wait()

    x = x_ref[...]
    h = lax.dot_general(x, w1b[sl].astype(jnp.bfloat16),
                        (((1,), (1,)), ((), ())),
                        preferred_element_type=jnp.float32)
    h = h + b1_ref[0]
    h = 0.5 * h * (1.0 + lax.erf(h * (1.0 / math.sqrt(2.0))))
    y = lax.dot_general(h.astype(jnp.bfloat16), w2b[sl].astype(jnp.bfloat16),
                        (((1,), (1,)), ((), ())),
                        preferred_element_type=jnp.float32)
    y_ref[...] = (y + b2_ref[0]).astype(jnp.bfloat16)


def _combine_ln_kernel(hid_ref, y0_ref, y1_ref, gates_ref, g_ref, b_ref, o_ref):
    i = pl.program_id(0)
    g = gates_ref[:, pl.ds(i * RB, RB)]
    rr = lax.broadcasted_iota(jnp.int32, (RB, RB), 0)
    cr = lax.broadcasted_iota(jnp.int32, (RB, RB), 1)
    ident = (rr == cr).astype(jnp.float32)
    gcols = lax.dot_general(ident, g, (((1,), (1,)), ((), ())),
                            preferred_element_type=jnp.float32)
    comb = (hid_ref[...] + gcols[:, 0:1] * y0_ref[...].astype(jnp.float32)
            + gcols[:, 1:2] * y1_ref[...].astype(jnp.float32))
    mu = jnp.mean(comb, axis=1, keepdims=True)
    d = comb - mu
    var = jnp.mean(d * d, axis=1, keepdims=True)
    o_ref[...] = d * lax.rsqrt(var + EPS) * g_ref[...] + b_ref[...]


HW = H // 2
CHUNK = 128
NCH = (2 * N) // 32 // CHUNK


@functools.cache
def _sc_kernels():
    mesh = plsc.VectorSubcoreMesh(core_axis_name="c", subcore_axis_name="s")

    @functools.partial(
        pl.kernel, mesh=mesh,
        out_type=jax.ShapeDtypeStruct((MAXPAD, HW), jnp.int32),
        scratch_types=[
            pltpu.VMEM((NCH, CHUNK), jnp.int32),
            pltpu.VMEM((CHUNK, HW), jnp.int32),
            pltpu.SemaphoreType.DMA,
        ],
    )
    def sc_dispatch(x_hbm, pos_hbm, xs_hbm, idx_v, rows_v, sem):
        wid = lax.axis_index("s") * 2 + lax.axis_index("c")
        pltpu.sync_copy(pos_hbm.at[wid], idx_v)
        row0 = (wid % 16) * 256
        for j in range(NCH):
            pltpu.sync_copy(x_hbm.at[pl.ds(row0 + j * CHUNK, CHUNK)], rows_v)
            pltpu.async_copy(rows_v, xs_hbm.at[idx_v.at[j]], sem).wait()

    @functools.partial(
        pl.kernel, mesh=mesh,
        out_type=jax.ShapeDtypeStruct((2 * N, HW), jnp.int32),
        scratch_types=[
            pltpu.VMEM((NCH, CHUNK), jnp.int32),
            pltpu.VMEM((CHUNK, HW), jnp.int32),
            pltpu.SemaphoreType.DMA,
        ],
    )
    def sc_combine_gather(ys_hbm, pos_hbm, out_hbm, idx_v, rows_v, sem):
        wid = lax.axis_index("s") * 2 + lax.axis_index("c")
        pltpu.sync_copy(pos_hbm.at[wid], idx_v)
        base = wid * 256
        for j in range(NCH):
            pltpu.async_copy(ys_hbm.at[idx_v.at[j]], rows_v, sem).wait()
            pltpu.sync_copy(rows_v, out_hbm.at[pl.ds(base + j * CHUNK, CHUNK)])

    return sc_dispatch, sc_combine_gather


def _bf16_to_i32(a):
    return lax.bitcast_convert_type(
        a.reshape(*a.shape[:-1], a.shape[-1] // 2, 2), jnp.int32)


def _i32_to_bf16(a):
    b = lax.bitcast_convert_type(a, jnp.bfloat16)
    return b.reshape(*b.shape[:-2], b.shape[-2] * 2)


def kernel(hidden_states, router_w, router_b, w1, b1, w2, b2, ln_g, ln_b):
    bsz, seqlen, hdim = hidden_states.shape
    x = hidden_states.reshape(N, H)

    pos, gates, be_full, nxt_full, sl_full, x_bf = pl.pallas_call(
        _router_plan_kernel,
        out_shape=[
            jax.ShapeDtypeStruct((2, N), jnp.int32),
            jax.ShapeDtypeStruct((2, N), jnp.float32),
            jax.ShapeDtypeStruct((8, 128), jnp.int32),
            jax.ShapeDtypeStruct((8, 128), jnp.int32),
            jax.ShapeDtypeStruct((8, 128), jnp.int32),
            jax.ShapeDtypeStruct((N, H), jnp.bfloat16),
        ],
    )(x, router_w, router_b.reshape(E, 1))

    pos3 = pos.reshape(32, NCH, CHUNK)

    sc_dispatch, sc_combine_gather = _sc_kernels()
    x_sorted = _i32_to_bf16(sc_dispatch(_bf16_to_i32(x_bf), pos3))

    be_vec = be_full.reshape(-1)[:MAX_BLOCKS]
    nxt_vec = nxt_full.reshape(-1)[:MAX_BLOCKS]
    sl_vec = sl_full.reshape(-1)[:MAX_BLOCKS]
    grid_spec = pltpu.PrefetchScalarGridSpec(
        num_scalar_prefetch=3,
        grid=(MAX_BLOCKS,),
        in_specs=[
            pl.BlockSpec((BLK, H), lambda i, be, nx, sl: (i, 0)),
            pl.BlockSpec(memory_space=pl.ANY),
            pl.BlockSpec((1, 1, F), lambda i, be, nx, sl: (be[i], 0, 0)),
            pl.BlockSpec(memory_space=pl.ANY),
            pl.BlockSpec((1, 1, H), lambda i, be, nx, sl: (be[i], 0, 0)),
        ],
        out_specs=pl.BlockSpec((BLK, H), lambda i, be, nx, sl: (i, 0)),
        scratch_shapes=[
            pltpu.VMEM((2, F, H), jnp.float32),
            pltpu.VMEM((2, H, F), jnp.float32),
            pltpu.SemaphoreType.DMA((2,)),
            pltpu.SemaphoreType.DMA((2,)),
        ],
    )
    y_sorted = pl.pallas_call(
        _expert_ffn_kernel,
        grid_spec=grid_spec,
        out_shape=jax.ShapeDtypeStruct((MAXPAD, H), jnp.bfloat16),
    )(be_vec, nxt_vec, sl_vec,
      x_sorted, w1, b1.reshape(E, 1, F), w2, b2.reshape(E, 1, H))

    y_tok = _i32_to_bf16(sc_combine_gather(_bf16_to_i32(y_sorted), pos3))

    out = pl.pallas_call(
        _combine_ln_kernel,
        grid=(N // RB,),
        in_specs=[
            pl.BlockSpec((RB, H), lambda i: (i, 0)),
            pl.BlockSpec((RB, H), lambda i: (i, 0)),
            pl.BlockSpec((RB, H), lambda i: (i + N // RB, 0)),
            pl.BlockSpec((2, N), lambda i: (0, 0)),
            pl.BlockSpec((1, H), lambda i: (0, 0)),
            pl.BlockSpec((1, H), lambda i: (0, 0)),
        ],
        out_specs=pl.BlockSpec((RB, H), lambda i: (i, 0)),
        out_shape=jax.ShapeDtypeStruct((N, H), jnp.float32),
    )(x, y_tok, y_tok, gates, ln_g.reshape(1, H), ln_b.reshape(1, H))

    return out.reshape(bsz, seqlen, hdim)


# speedup vs baseline: 4.3561x; 4.3561x over previous
import functools
import math

import jax
import jax.numpy as jnp
from jax import lax
from jax.experimental import pallas as pl
from jax.experimental.pallas import tpu as pltpu
from jax.experimental.pallas import tpu_sc as plsc

N = 4096
H = 1024
HW = H // 2
F = 2048
E = 8
BLK = 256
MAX_BLOCKS = (2 * N) // BLK + (E - 1)
MAXPAD = MAX_BLOCKS * BLK
CB = 512
RB = 512
EPS = 1e-5

NEG = -1e30


def _pack_halves(lo_bf, hi_bf):
    lo = lax.bitcast_convert_type(lo_bf, jnp.uint16).astype(jnp.uint32)
    hi = lax.bitcast_convert_type(hi_bf, jnp.uint16).astype(jnp.uint32)
    return lax.bitcast_convert_type(lo | (hi << 16), jnp.int32)


def _unpack_halves(words_i32):
    w = lax.bitcast_convert_type(words_i32, jnp.uint32)
    lo = lax.bitcast_convert_type((w & 0xFFFF).astype(jnp.uint16),
                                  jnp.bfloat16)
    hi = lax.bitcast_convert_type((w >> 16).astype(jnp.uint16), jnp.bfloat16)
    return lo, hi

def _router_plan_kernel(x_ref, rw_ref, rb_ref, pos_ref, gates_ref, be_ref,
                        nxt_ref, sl_ref, xb_ref):
    x = x_ref[...]
    xbf = x.astype(jnp.bfloat16)
    xb_ref[...] = _pack_halves(xbf[:, :HW], xbf[:, HW:])
    rw = rw_ref[...]
    rb = rb_ref[...]
    logits = lax.dot_general(rw, x, (((1,), (1,)), ((), ())),
                             preferred_element_type=jnp.float32)
    logits = logits + rb

    r8 = lax.broadcasted_iota(jnp.int32, (E, E), 0)
    c8 = lax.broadcasted_iota(jnp.int32, (E, E), 1)
    lt8_incl = (r8 >= c8).astype(jnp.float32)
    lt8_excl = (r8 > c8).astype(jnp.float32)

    def first_argmax_onehot(lg):
        m = jnp.max(lg, axis=0, keepdims=True)
        eq = (lg == m).astype(jnp.float32)
        csum = lax.dot_general(lt8_incl, eq, (((1,), (0,)), ((), ())),
                               preferred_element_type=jnp.float32)
        oh = eq * (csum == 1.0).astype(jnp.float32)
        return m, oh

    m1, oh0 = first_argmax_onehot(logits)
    logits2 = jnp.where(oh0 > 0.0, NEG, logits)
    m2, oh1 = first_argmax_onehot(logits2)

    g0 = 1.0 / (1.0 + jnp.exp(m2 - m1))
    g1 = 1.0 - g0

    rc = lax.broadcasted_iota(jnp.int32, (CB, CB), 0)
    cc = lax.broadcasted_iota(jnp.int32, (CB, CB), 1)
    ut = (rc < cc).astype(jnp.float32)

    def ranks(oh):
        parts = []
        run = jnp.zeros((E, 1), jnp.float32)
        for cb in range(N // CB):
            ohb = oh[:, cb * CB:(cb + 1) * CB]
            loc = lax.dot_general(ohb, ut, (((1,), (0,)), ((), ())),
                                  preferred_element_type=jnp.float32)
            rank = loc + run
            parts.append(jnp.sum(rank * ohb, axis=0, keepdims=True))
            run = run + jnp.sum(ohb, axis=1, keepdims=True)
        return jnp.concatenate(parts, axis=1), run

    r0, cnt0 = ranks(oh0)
    r1, cnt1 = ranks(oh1)

    counts = cnt0 + cnt1
    nblk = jnp.floor((counts + (BLK - 1)) / BLK)
    blk_start = lax.dot_general(lt8_excl, nblk, (((1,), (0,)), ((), ())),
                                preferred_element_type=jnp.float32)
    padded_off = blk_start * BLK

    pos0 = r0 + jnp.sum(padded_off * oh0, axis=0, keepdims=True)
    pos1 = r1 + jnp.sum((padded_off + cnt0) * oh1, axis=0, keepdims=True)
    pos_ref[...] = jnp.concatenate([pos0, pos1], axis=0).astype(jnp.int32)
    gates_ref[...] = jnp.concatenate([g0, g1], axis=0)

    cum_end = blk_start + nblk
    nonempty = nblk > 0.0
    e_col = lax.broadcasted_iota(jnp.int32, (E, 1), 0).astype(jnp.float32)
    last_e = jnp.max(jnp.where(nonempty, e_col, 0.0), axis=0, keepdims=True)

    b_iota = lax.broadcasted_iota(jnp.int32, (8, 128), 1).astype(jnp.float32)
    be = jnp.zeros((8, 128), jnp.float32)
    sl = jnp.zeros((8, 128), jnp.float32)
    for e in range(E):
        past_end = (b_iota >= cum_end[e:e + 1, 0:1]).astype(jnp.float32)
        be = be + past_end
        sl = sl + jnp.where(nonempty[e:e + 1, 0:1], past_end, 0.0)
    be = jnp.minimum(be, last_e)
    sl = sl - 2.0 * jnp.floor(sl * 0.5)

    nxt_for = [None] * E
    nxt_for[E - 1] = jnp.zeros((1, 1), jnp.float32) + float(E - 1)
    for e in range(E - 2, -1, -1):
        cand = nxt_for[e + 1]
        cand = jnp.where(cand == float(e + 1), float(e), cand)
        nxt_for[e] = jnp.where(nonempty[e + 1:e + 2, 0:1], float(e + 1), cand)
    nxt = jnp.zeros((8, 128), jnp.float32)
    for e in range(E):
        in_region = ((b_iota >= blk_start[e:e + 1, 0:1])
                     & (b_iota < cum_end[e:e + 1, 0:1])).astype(jnp.float32)
        nxt = nxt + nxt_for[e] * in_region

    be_ref[...] = be.astype(jnp.int32)
    nxt_ref[...] = nxt.astype(jnp.int32)
    sl_ref[...] = sl.astype(jnp.int32)


def _expert_ffn_kernel(be_ref, nxt_ref, sl_ref, x_ref, w1_hbm, b1_ref,
                       w2_hbm, b2_ref, y_ref, w1b, w2b, s1, s2):
    i = pl.program_id(0)
    e = be_ref[i]
    sl = sl_ref[i]
    nxt = nxt_ref[i]
    prev_e = be_ref[jnp.maximum(i - 1, 0)]
    boundary = jnp.logical_or(i == 0, e != prev_e)

    @pl.when(i == 0)
    def _():
        pltpu.make_async_copy(w1_hbm.at[e], w1b.at[sl], s1.at[sl]).start()
        pltpu.make_async_copy(w2_hbm.at[e], w2b.at[sl], s2.at[sl]).start()

    @pl.when(jnp.logical_and(boundary, nxt != e))
    def _():
        nsl = 1 - sl
        pltpu.make_async_copy(w1_hbm.at[nxt], w1b.at[nsl], s1.at[nsl]).start()
        pltpu.make_async_copy(w2_hbm.at[nxt], w2b.at[nsl], s2.at[nsl]).start()

    @pl.when(boundary)
    def _():
        pltpu.make_async_copy(w1_hbm.at[e], w1b.at[sl], s1.at[sl]).wait()
        pltpu.make_async_copy(w2_hbm.at[e], w2b.at[sl], s2.at[sl]).wait()

    x_lo, x_hi = _unpack_halves(x_ref[...])
    w1c = w1b[sl]
    h = lax.dot_general(x_lo, w1c[:, :HW].astype(jnp.bfloat16),
                        (((1,), (1,)), ((), ())),
                        preferred_element_type=jnp.float32)
    h = h + lax.dot_general(x_hi, w1c[:, HW:].astype(jnp.bfloat16),
                            (((1,), (1,)), ((), ())),
                            preferred_element_type=jnp.float32)
    h = h + b1_ref[0]
    h = 0.5 * h * (1.0 + lax.erf(h * (1.0 / math.sqrt(2.0))))
    y = lax.dot_general(h.astype(jnp.bfloat16), w2b[sl].astype(jnp.bfloat16),
                        (((1,), (1,)), ((), ())),
                        preferred_element_type=jnp.float32)
    y = y + b2_ref[0]
    ybf = y.astype(jnp.bfloat16)
    y_ref[...] = _pack_halves(ybf[:, :HW], ybf[:, HW:])


def _combine_ln_kernel(hid_ref, y0_ref, y1_ref, gates_ref, g_ref, b_ref, o_ref):
    i = pl.program_id(0)
    g = gates_ref[:, pl.ds(i * RB, RB)]
    rr = lax.broadcasted_iota(jnp.int32, (RB, RB), 0)
    cr = lax.broadcasted_iota(jnp.int32, (RB, RB), 1)
    ident = (rr == cr).astype(jnp.float32)
    gcols = lax.dot_general(ident, g, (((1,), (1,)), ((), ())),
                            preferred_element_type=jnp.float32)
    g0 = gcols[:, 0:1]
    g1 = gcols[:, 1:2]
    y0_lo, y0_hi = _unpack_halves(y0_ref[...])
    y1_lo, y1_hi = _unpack_halves(y1_ref[...])
    hid = hid_ref[...]
    c_lo = (hid[:, :HW] + g0 * y0_lo.astype(jnp.float32)
            + g1 * y1_lo.astype(jnp.float32))
    c_hi = (hid[:, HW:] + g0 * y0_hi.astype(jnp.float32)
            + g1 * y1_hi.astype(jnp.float32))
    s = (jnp.sum(c_lo, axis=1, keepdims=True)
         + jnp.sum(c_hi, axis=1, keepdims=True))
    mu = s * (1.0 / H)
    d_lo = c_lo - mu
    d_hi = c_hi - mu
    v = (jnp.sum(d_lo * d_lo, axis=1, keepdims=True)
         + jnp.sum(d_hi * d_hi, axis=1, keepdims=True))
    rs = lax.rsqrt(v * (1.0 / H) + EPS)
    o_ref[:, :HW] = d_lo * rs * g_ref[:, :HW] + b_ref[:, :HW]
    o_ref[:, HW:] = d_hi * rs * g_ref[:, HW:] + b_ref[:, HW:]


CHUNK = 128
NCH = (2 * N) // 32 // CHUNK


@functools.cache
def _sc_kernels():
    mesh = plsc.VectorSubcoreMesh(core_axis_name="c", subcore_axis_name="s")

    @functools.partial(
        pl.kernel, mesh=mesh,
        out_type=jax.ShapeDtypeStruct((MAXPAD, HW), jnp.int32),
        scratch_types=[
            pltpu.VMEM((NCH, CHUNK), jnp.int32),
            pltpu.VMEM((CHUNK, HW), jnp.int32),
            pltpu.SemaphoreType.DMA,
        ],
    )
    def sc_dispatch(x_hbm, pos_hbm, xs_hbm, idx_v, rows_v, sem):
        wid = lax.axis_index("s") * 2 + lax.axis_index("c")
        pltpu.sync_copy(pos_hbm.at[wid], idx_v)
        row0 = (wid % 16) * 256
        for j in range(NCH):
            pltpu.sync_copy(x_hbm.at[pl.ds(row0 + j * CHUNK, CHUNK)], rows_v)
            pltpu.async_copy(rows_v, xs_hbm.at[idx_v.at[j]], sem).wait()

    @functools.partial(
        pl.kernel, mesh=mesh,
        out_type=jax.ShapeDtypeStruct((2 * N, HW), jnp.int32),
        scratch_types=[
            pltpu.VMEM((NCH, CHUNK), jnp.int32),
            pltpu.VMEM((CHUNK, HW), jnp.int32),
            pltpu.SemaphoreType.DMA,
        ],
    )
    def sc_combine_gather(ys_hbm, pos_hbm, out_hbm, idx_v, rows_v, sem):
        wid = lax.axis_index("s") * 2 + lax.axis_index("c")
        pltpu.sync_copy(pos_hbm.at[wid], idx_v)
        base = wid * 256
        for j in range(NCH):
            pltpu.async_copy(ys_hbm.at[idx_v.at[j]], rows_v, sem).wait()
            pltpu.sync_copy(rows_v, out_hbm.at[pl.ds(base + j * CHUNK, CHUNK)])

    return sc_dispatch, sc_combine_gather


def kernel(hidden_states, router_w, router_b, w1, b1, w2, b2, ln_g, ln_b):
    bsz, seqlen, hdim = hidden_states.shape
    x = hidden_states.reshape(N, H)

    pos, gates, be_full, nxt_full, sl_full, x_bf = pl.pallas_call(
        _router_plan_kernel,
        out_shape=[
            jax.ShapeDtypeStruct((2, N), jnp.int32),
            jax.ShapeDtypeStruct((2, N), jnp.float32),
            jax.ShapeDtypeStruct((8, 128), jnp.int32),
            jax.ShapeDtypeStruct((8, 128), jnp.int32),
            jax.ShapeDtypeStruct((8, 128), jnp.int32),
            jax.ShapeDtypeStruct((N, HW), jnp.int32),
        ],
    )(x, router_w, router_b.reshape(E, 1))

    pos3 = pos.reshape(32, NCH, CHUNK)

    sc_dispatch, sc_combine_gather = _sc_kernels()
    x_sorted = sc_dispatch(x_bf, pos3)

    be_vec = be_full.reshape(-1)[:MAX_BLOCKS]
    nxt_vec = nxt_full.reshape(-1)[:MAX_BLOCKS]
    sl_vec = sl_full.reshape(-1)[:MAX_BLOCKS]
    grid_spec = pltpu.PrefetchScalarGridSpec(
        num_scalar_prefetch=3,
        grid=(MAX_BLOCKS,),
        in_specs=[
            pl.BlockSpec((BLK, HW), lambda i, be, nx, sl: (i, 0)),
            pl.BlockSpec(memory_space=pl.ANY),
            pl.BlockSpec((1, 1, F), lambda i, be, nx, sl: (be[i], 0, 0)),
            pl.BlockSpec(memory_space=pl.ANY),
            pl.BlockSpec((1, 1, H), lambda i, be, nx, sl: (be[i], 0, 0)),
        ],
        out_specs=pl.BlockSpec((BLK, HW), lambda i, be, nx, sl: (i, 0)),
        scratch_shapes=[
            pltpu.VMEM((2, F, H), jnp.float32),
            pltpu.VMEM((2, H, F), jnp.float32),
            pltpu.SemaphoreType.DMA((2,)),
            pltpu.SemaphoreType.DMA((2,)),
        ],
    )
    y_sorted = pl.pallas_call(
        _expert_ffn_kernel,
        grid_spec=grid_spec,
        out_shape=jax.ShapeDtypeStruct((MAXPAD, HW), jnp.int32),
    )(be_vec, nxt_vec, sl_vec,
      x_sorted, w1, b1.reshape(E, 1, F), w2, b2.reshape(E, 1, H))

    y_tok = sc_combine_gather(y_sorted, pos3)

    out = pl.pallas_call(
        _combine_ln_kernel,
        grid=(N // RB,),
        in_specs=[
            pl.BlockSpec((RB, H), lambda i: (i, 0)),
            pl.BlockSpec((RB, HW), lambda i: (i, 0)),
            pl.BlockSpec((RB, HW), lambda i: (i + N // RB, 0)),
            pl.BlockSpec((2, N), lambda i: (0, 0)),
            pl.BlockSpec((1, H), lambda i: (0, 0)),
            pl.BlockSpec((1, H), lambda i: (0, 0)),
        ],
        out_specs=pl.BlockSpec((RB, H), lambda i: (i, 0)),
        out_shape=jax.ShapeDtypeStruct((N, H), jnp.float32),
    )(x, y_tok, y_tok, gates, ln_g.reshape(1, H), ln_b.reshape(1, H))

    return out.reshape(bsz, seqlen, hdim)


# speedup vs baseline: 4.6583x; 1.0694x over previous
import functools
import math

import jax
import jax.numpy as jnp
from jax import lax
from jax.experimental import pallas as pl
from jax.experimental.pallas import tpu as pltpu
from jax.experimental.pallas import tpu_sc as plsc

N = 4096
H = 1024
HW = H // 2
F = 2048
E = 8
BLK = 256
MAX_BLOCKS = (2 * N) // BLK + (E - 1)
MAXPAD = MAX_BLOCKS * BLK
CB = 512
RB = 512
EPS = 1e-5

NEG = -1e30


def _pack_halves(lo_bf, hi_bf):
    lo = lax.bitcast_convert_type(lo_bf, jnp.uint16).astype(jnp.uint32)
    hi = lax.bitcast_convert_type(hi_bf, jnp.uint16).astype(jnp.uint32)
    return lax.bitcast_convert_type(lo | (hi << 16), jnp.int32)


def _unpack_halves(words_i32):
    w = lax.bitcast_convert_type(words_i32, jnp.uint32)
    lo = lax.bitcast_convert_type((w & 0xFFFF).astype(jnp.uint16),
                                  jnp.bfloat16)
    hi = lax.bitcast_convert_type((w >> 16).astype(jnp.uint16), jnp.bfloat16)
    return lo, hi

def _router_plan_kernel(x_ref, rw_ref, rb_ref, pos_ref, gates_ref, be_ref,
                        nxt_ref, sl_ref, vld_ref, xb_ref):
    x = x_ref[...]
    xbf = x.astype(jnp.bfloat16)
    xb_ref[...] = _pack_halves(xbf[:, :HW], xbf[:, HW:])
    rw = rw_ref[...]
    rb = rb_ref[...]
    logits = lax.dot_general(rw, x, (((1,), (1,)), ((), ())),
                             preferred_element_type=jnp.float32)
    logits = logits + rb

    r8 = lax.broadcasted_iota(jnp.int32, (E, E), 0)
    c8 = lax.broadcasted_iota(jnp.int32, (E, E), 1)
    lt8_incl = (r8 >= c8).astype(jnp.float32)
    lt8_excl = (r8 > c8).astype(jnp.float32)

    def first_argmax_onehot(lg):
        m = jnp.max(lg, axis=0, keepdims=True)
        eq = (lg == m).astype(jnp.float32)
        csum = lax.dot_general(lt8_incl, eq, (((1,), (0,)), ((), ())),
                               preferred_element_type=jnp.float32)
        oh = eq * (csum == 1.0).astype(jnp.float32)
        return m, oh

    m1, oh0 = first_argmax_onehot(logits)
    logits2 = jnp.where(oh0 > 0.0, NEG, logits)
    m2, oh1 = first_argmax_onehot(logits2)

    g0 = 1.0 / (1.0 + jnp.exp(m2 - m1))
    g1 = 1.0 - g0

    rc = lax.broadcasted_iota(jnp.int32, (CB, CB), 0)
    cc = lax.broadcasted_iota(jnp.int32, (CB, CB), 1)
    ut = (rc < cc).astype(jnp.float32)

    def ranks(oh):
        parts = []
        run = jnp.zeros((E, 1), jnp.float32)
        for cb in range(N // CB):
            ohb = oh[:, cb * CB:(cb + 1) * CB]
            loc = lax.dot_general(ohb, ut, (((1,), (0,)), ((), ())),
                                  preferred_element_type=jnp.float32)
            rank = loc + run
            parts.append(jnp.sum(rank * ohb, axis=0, keepdims=True))
            run = run + jnp.sum(ohb, axis=1, keepdims=True)
        return jnp.concatenate(parts, axis=1), run

    r0, cnt0 = ranks(oh0)
    r1, cnt1 = ranks(oh1)

    counts = cnt0 + cnt1
    nblk = jnp.floor((counts + (BLK - 1)) / BLK)
    blk_start = lax.dot_general(lt8_excl, nblk, (((1,), (0,)), ((), ())),
                                preferred_element_type=jnp.float32)
    padded_off = blk_start * BLK

    pos0 = r0 + jnp.sum(padded_off * oh0, axis=0, keepdims=True)
    pos1 = r1 + jnp.sum((padded_off + cnt0) * oh1, axis=0, keepdims=True)
    pos_ref[...] = jnp.concatenate([pos0, pos1], axis=0).astype(jnp.int32)
    gates_ref[...] = jnp.concatenate([g0, g1], axis=0)

    cum_end = blk_start + nblk
    nonempty = nblk > 0.0
    e_col = lax.broadcasted_iota(jnp.int32, (E, 1), 0).astype(jnp.float32)
    last_e = jnp.max(jnp.where(nonempty, e_col, 0.0), axis=0, keepdims=True)

    b_iota = lax.broadcasted_iota(jnp.int32, (8, 128), 1).astype(jnp.float32)
    be = jnp.zeros((8, 128), jnp.float32)
    sl = jnp.zeros((8, 128), jnp.float32)
    for e in range(E):
        past_end = (b_iota >= cum_end[e:e + 1, 0:1]).astype(jnp.float32)
        be = be + past_end
        sl = sl + jnp.where(nonempty[e:e + 1, 0:1], past_end, 0.0)
    be = jnp.minimum(be, last_e)
    sl = sl - 2.0 * jnp.floor(sl * 0.5)

    nxt_for = [None] * E
    nxt_for[E - 1] = jnp.zeros((1, 1), jnp.float32) + float(E - 1)
    for e in range(E - 2, -1, -1):
        cand = nxt_for[e + 1]
        cand = jnp.where(cand == float(e + 1), float(e), cand)
        nxt_for[e] = jnp.where(nonempty[e + 1:e + 2, 0:1], float(e + 1), cand)
    nxt = jnp.zeros((8, 128), jnp.float32)
    for e in range(E):
        in_region = ((b_iota >= blk_start[e:e + 1, 0:1])
                     & (b_iota < cum_end[e:e + 1, 0:1])).astype(jnp.float32)
        nxt = nxt + nxt_for[e] * in_region

    be_ref[...] = be.astype(jnp.int32)
    nxt_ref[...] = nxt.astype(jnp.int32)
    sl_ref[...] = sl.astype(jnp.int32)
    vld_ref[...] = (b_iota < cum_end[E - 1:E, 0:1]).astype(jnp.int32)


def _expert_ffn_kernel(be_ref, nxt_ref, sl_ref, vld_ref, x_ref, w1_hbm,
                       b1_ref, w2_hbm, b2_ref, y_ref, w1b, w2b, s1, s2):
    i = pl.program_id(0)
    e = be_ref[i]
    sl = sl_ref[i]
    nxt = nxt_ref[i]
    prev_e = be_ref[jnp.maximum(i - 1, 0)]
    boundary = jnp.logical_or(i == 0, e != prev_e)

    @pl.when(i == 0)
    def _():
        pltpu.make_async_copy(w1_hbm.at[e], w1b.at[sl], s1.at[sl]).start()
        pltpu.make_async_copy(w2_hbm.at[e], w2b.at[sl], s2.at[sl]).start()

    @pl.when(jnp.logical_and(boundary, nxt != e))
    def _():
        nsl = 1 - sl
        pltpu.make_async_copy(w1_hbm.at[nxt], w1b.at[nsl], s1.at[nsl]).start()
        pltpu.make_async_copy(w2_hbm.at[nxt], w2b.at[nsl], s2.at[nsl]).start()

    @pl.when(boundary)
    def _():
        pltpu.make_async_copy(w1_hbm.at[e], w1b.at[sl], s1.at[sl]).wait()
        pltpu.make_async_copy(w2_hbm.at[e], w2b.at[sl], s2.at[sl]).wait()

    @pl.when(vld_ref[i] > 0)
    def _():
        x_lo, x_hi = _unpack_halves(x_ref[...])
        w1c = w1b[sl]
        h = lax.dot_general(x_lo, w1c[:, :HW].astype(jnp.bfloat16),
                            (((1,), (1,)), ((), ())),
                            preferred_element_type=jnp.float32)
        h = h + lax.dot_general(x_hi, w1c[:, HW:].astype(jnp.bfloat16),
                                (((1,), (1,)), ((), ())),
                                preferred_element_type=jnp.float32)
        h = h + b1_ref[0]
        h = 0.5 * h * (1.0 + lax.erf(h * (1.0 / math.sqrt(2.0))))
        y = lax.dot_general(h.astype(jnp.bfloat16),
                            w2b[sl].astype(jnp.bfloat16),
                            (((1,), (1,)), ((), ())),
                            preferred_element_type=jnp.float32)
        y = y + b2_ref[0]
        ybf = y.astype(jnp.bfloat16)
        y_ref[...] = _pack_halves(ybf[:, :HW], ybf[:, HW:])


def _combine_ln_kernel(hid_ref, y0_ref, y1_ref, gates_ref, g_ref, b_ref, o_ref):
    i = pl.program_id(0)
    g = gates_ref[:, pl.ds(i * RB, RB)]
    rr = lax.broadcasted_iota(jnp.int32, (RB, RB), 0)
    cr = lax.broadcasted_iota(jnp.int32, (RB, RB), 1)
    ident = (rr == cr).astype(jnp.float32)
    gcols = lax.dot_general(ident, g, (((1,), (1,)), ((), ())),
                            preferred_element_type=jnp.float32)
    g0 = gcols[:, 0:1]
    g1 = gcols[:, 1:2]
    y0_lo, y0_hi = _unpack_halves(y0_ref[...])
    y1_lo, y1_hi = _unpack_halves(y1_ref[...])
    hid = hid_ref[...]
    c_lo = (hid[:, :HW] + g0 * y0_lo.astype(jnp.float32)
            + g1 * y1_lo.astype(jnp.float32))
    c_hi = (hid[:, HW:] + g0 * y0_hi.astype(jnp.float32)
            + g1 * y1_hi.astype(jnp.float32))
    s = (jnp.sum(c_lo, axis=1, keepdims=True)
         + jnp.sum(c_hi, axis=1, keepdims=True))
    mu = s * (1.0 / H)
    d_lo = c_lo - mu
    d_hi = c_hi - mu
    v = (jnp.sum(d_lo * d_lo, axis=1, keepdims=True)
         + jnp.sum(d_hi * d_hi, axis=1, keepdims=True))
    rs = lax.rsqrt(v * (1.0 / H) + EPS)
    o_ref[:, :HW] = d_lo * rs * g_ref[:, :HW] + b_ref[:, :HW]
    o_ref[:, HW:] = d_hi * rs * g_ref[:, HW:] + b_ref[:, HW:]


CHUNK = 64
NCH = (2 * N) // 32 // CHUNK


@functools.cache
def _sc_kernels():
    mesh = plsc.VectorSubcoreMesh(core_axis_name="c", subcore_axis_name="s")

    @functools.partial(
        pl.kernel, mesh=mesh,
        out_type=jax.ShapeDtypeStruct((MAXPAD, HW), jnp.int32),
        scratch_types=[
            pltpu.VMEM((2, 128), jnp.int32),
            pltpu.VMEM((128, HW), jnp.int32),
            pltpu.SemaphoreType.DMA,
            pltpu.SemaphoreType.DMA,
        ],
    )
    def sc_dispatch(x_hbm, pos_hbm, xs_hbm, idx_v, rows_v, s0, s1):
        wid = lax.axis_index("s") * 2 + lax.axis_index("c")
        pltpu.sync_copy(pos_hbm.at[wid], idx_v)
        pltpu.sync_copy(x_hbm.at[pl.ds(wid * 128, 128)], rows_v)
        c0 = pltpu.async_copy(rows_v, xs_hbm.at[idx_v.at[0]], s0)
        c1 = pltpu.async_copy(rows_v, xs_hbm.at[idx_v.at[1]], s1)
        c0.wait()
        c1.wait()

    @functools.partial(
        pl.kernel, mesh=mesh,
        out_type=jax.ShapeDtypeStruct((2 * N, HW), jnp.int32),
        scratch_types=[
            pltpu.VMEM((NCH, CHUNK), jnp.int32),
            pltpu.VMEM((2, CHUNK, HW), jnp.int32),
            pltpu.SemaphoreType.DMA,
            pltpu.SemaphoreType.DMA,
        ],
    )
    def sc_combine_gather(ys_hbm, pos_hbm, out_hbm, idx_v, rows_v, s0, s1):
        wid = lax.axis_index("s") * 2 + lax.axis_index("c")
        pltpu.sync_copy(pos_hbm.at[wid], idx_v)
        base = wid * 256
        sems = [s0, s1]
        copies = [None, None]
        copies[0] = pltpu.async_copy(ys_hbm.at[idx_v.at[0]], rows_v.at[0],
                                     sems[0])
        for j in range(NCH):
            b = j % 2
            if j + 1 < NCH:
                copies[1 - b] = pltpu.async_copy(
                    ys_hbm.at[idx_v.at[j + 1]], rows_v.at[1 - b],
                    sems[1 - b])
            copies[b].wait()
            pltpu.sync_copy(rows_v.at[b],
                            out_hbm.at[pl.ds(base + j * CHUNK, CHUNK)])

    return sc_dispatch, sc_combine_gather


def kernel(hidden_states, router_w, router_b, w1, b1, w2, b2, ln_g, ln_b):
    bsz, seqlen, hdim = hidden_states.shape
    x = hidden_states.reshape(N, H)

    pos, gates, be_full, nxt_full, sl_full, vld_full, x_bf = pl.pallas_call(
        _router_plan_kernel,
        out_shape=[
            jax.ShapeDtypeStruct((2, N), jnp.int32),
            jax.ShapeDtypeStruct((2, N), jnp.float32),
            jax.ShapeDtypeStruct((8, 128), jnp.int32),
            jax.ShapeDtypeStruct((8, 128), jnp.int32),
            jax.ShapeDtypeStruct((8, 128), jnp.int32),
            jax.ShapeDtypeStruct((8, 128), jnp.int32),
            jax.ShapeDtypeStruct((N, HW), jnp.int32),
        ],
    )(x, router_w, router_b.reshape(E, 1))

    pos3 = pos.reshape(32, NCH, CHUNK)
    pos_t = pos.reshape(2, 32, 128).transpose(1, 0, 2)

    sc_dispatch, sc_combine_gather = _sc_kernels()
    x_sorted = sc_dispatch(x_bf, pos_t)

    be_vec = be_full.reshape(-1)[:MAX_BLOCKS]
    nxt_vec = nxt_full.reshape(-1)[:MAX_BLOCKS]
    sl_vec = sl_full.reshape(-1)[:MAX_BLOCKS]
    vld_vec = vld_full.reshape(-1)[:MAX_BLOCKS]
    grid_spec = pltpu.PrefetchScalarGridSpec(
        num_scalar_prefetch=4,
        grid=(MAX_BLOCKS,),
        in_specs=[
            pl.BlockSpec((BLK, HW), lambda i, be, nx, sl, v: (i, 0)),
            pl.BlockSpec(memory_space=pl.ANY),
            pl.BlockSpec((1, 1, F), lambda i, be, nx, sl, v: (be[i], 0, 0)),
            pl.BlockSpec(memory_space=pl.ANY),
            pl.BlockSpec((1, 1, H), lambda i, be, nx, sl, v: (be[i], 0, 0)),
        ],
        out_specs=pl.BlockSpec((BLK, HW), lambda i, be, nx, sl, v: (i, 0)),
        scratch_shapes=[
            pltpu.VMEM((2, F, H), jnp.float32),
            pltpu.VMEM((2, H, F), jnp.float32),
            pltpu.SemaphoreType.DMA((2,)),
            pltpu.SemaphoreType.DMA((2,)),
        ],
    )
    y_sorted = pl.pallas_call(
        _expert_ffn_kernel,
        grid_spec=grid_spec,
        out_shape=jax.ShapeDtypeStruct((MAXPAD, HW), jnp.int32),
    )(be_vec, nxt_vec, sl_vec, vld_vec,
      x_sorted, w1, b1.reshape(E, 1, F), w2, b2.reshape(E, 1, H))

    y_tok = sc_combine_gather(y_sorted, pos3)

    out = pl.pallas_call(
        _combine_ln_kernel,
        grid=(N // RB,),
        in_specs=[
            pl.BlockSpec((RB, H), lambda i: (i, 0)),
            pl.BlockSpec((RB, HW), lambda i: (i, 0)),
            pl.BlockSpec((RB, HW), lambda i: (i + N // RB, 0)),
            pl.BlockSpec((2, N), lambda i: (0, 0)),
            pl.BlockSpec((1, H), lambda i: (0, 0)),
            pl.BlockSpec((1, H), lambda i: (0, 0)),
        ],
        out_specs=pl.BlockSpec((RB, H), lambda i: (i, 0)),
        out_shape=jax.ShapeDtypeStruct((N, H), jnp.float32),
    )(x, y_tok, y_tok, gates, ln_g.reshape(1, H), ln_b.reshape(1, H))

    return out.reshape(bsz, seqlen, hdim)


# speedup vs baseline: 4.6712x; 1.0028x over previous
import functools
import math

import jax
import jax.numpy as jnp
from jax import lax
from jax.experimental import pallas as pl
from jax.experimental.pallas import tpu as pltpu
from jax.experimental.pallas import tpu_sc as plsc

N = 4096
H = 1024
HW = H // 2
F = 2048
E = 8
BLK = 256
MAX_BLOCKS = (2 * N) // BLK + (E - 1)
MAXPAD = MAX_BLOCKS * BLK
CB = 512
RB = 512
EPS = 1e-5

NEG = -1e30


def _pack_halves(lo_bf, hi_bf):
    lo = lax.bitcast_convert_type(lo_bf, jnp.uint16).astype(jnp.uint32)
    hi = lax.bitcast_convert_type(hi_bf, jnp.uint16).astype(jnp.uint32)
    return lax.bitcast_convert_type(lo | (hi << 16), jnp.int32)


def _unpack_halves(words_i32):
    w = lax.bitcast_convert_type(words_i32, jnp.uint32)
    lo = lax.bitcast_convert_type((w & 0xFFFF).astype(jnp.uint16),
                                  jnp.bfloat16)
    hi = lax.bitcast_convert_type((w >> 16).astype(jnp.uint16), jnp.bfloat16)
    return lo, hi

def _router_plan_kernel(x_ref, rw_ref, rb_ref, pos_ref, gates_ref, be_ref,
                        nxt_ref, sl_ref, vld_ref, xb_ref, pos_t_ref):
    x = x_ref[...]
    xbf = x.astype(jnp.bfloat16)
    xb_ref[...] = _pack_halves(xbf[:, :HW], xbf[:, HW:])
    rw = rw_ref[...]
    rb = rb_ref[...]
    logits = lax.dot_general(rw, x, (((1,), (1,)), ((), ())),
                             preferred_element_type=jnp.float32)
    logits = logits + rb

    r8 = lax.broadcasted_iota(jnp.int32, (E, E), 0)
    c8 = lax.broadcasted_iota(jnp.int32, (E, E), 1)
    lt8_incl = (r8 >= c8).astype(jnp.float32)
    lt8_excl = (r8 > c8).astype(jnp.float32)

    def first_argmax_onehot(lg):
        m = jnp.max(lg, axis=0, keepdims=True)
        eq = (lg == m).astype(jnp.float32)
        csum = lax.dot_general(lt8_incl, eq, (((1,), (0,)), ((), ())),
                               preferred_element_type=jnp.float32)
        oh = eq * (csum == 1.0).astype(jnp.float32)
        return m, oh

    m1, oh0 = first_argmax_onehot(logits)
    logits2 = jnp.where(oh0 > 0.0, NEG, logits)
    m2, oh1 = first_argmax_onehot(logits2)

    g0 = 1.0 / (1.0 + jnp.exp(m2 - m1))
    g1 = 1.0 - g0

    rc = lax.broadcasted_iota(jnp.int32, (CB, CB), 0)
    cc = lax.broadcasted_iota(jnp.int32, (CB, CB), 1)
    ut = (rc < cc).astype(jnp.float32)

    def ranks(oh):
        parts = []
        run = jnp.zeros((E, 1), jnp.float32)
        for cb in range(N // CB):
            ohb = oh[:, cb * CB:(cb + 1) * CB]
            loc = lax.dot_general(ohb, ut, (((1,), (0,)), ((), ())),
                                  preferred_element_type=jnp.float32)
            rank = loc + run
            parts.append(jnp.sum(rank * ohb, axis=0, keepdims=True))
            run = run + jnp.sum(ohb, axis=1, keepdims=True)
        return jnp.concatenate(parts, axis=1), run

    r0, cnt0 = ranks(oh0)
    r1, cnt1 = ranks(oh1)

    counts = cnt0 + cnt1
    nblk = jnp.floor((counts + (BLK - 1)) / BLK)
    blk_start = lax.dot_general(lt8_excl, nblk, (((1,), (0,)), ((), ())),
                                preferred_element_type=jnp.float32)
    padded_off = blk_start * BLK

    pos0 = r0 + jnp.sum(padded_off * oh0, axis=0, keepdims=True)
    pos1 = r1 + jnp.sum((padded_off + cnt0) * oh1, axis=0, keepdims=True)
    pos_ref[...] = jnp.concatenate([pos0, pos1], axis=0).astype(jnp.int32)
    gates_ref[...] = jnp.concatenate([g0, g1], axis=0)
    p0i = pos0.astype(jnp.int32)
    p1i = pos1.astype(jnp.int32)
    for w in range(32):
        pos_t_ref[w, 0:1, :] = p0i[:, w * 128:(w + 1) * 128]
        pos_t_ref[w, 1:2, :] = p1i[:, w * 128:(w + 1) * 128]

    cum_end = blk_start + nblk
    nonempty = nblk > 0.0
    e_col = lax.broadcasted_iota(jnp.int32, (E, 1), 0).astype(jnp.float32)
    last_e = jnp.max(jnp.where(nonempty, e_col, 0.0), axis=0, keepdims=True)

    b_iota = lax.broadcasted_iota(jnp.int32, (8, 128), 1).astype(jnp.float32)
    be = jnp.zeros((8, 128), jnp.float32)
    sl = jnp.zeros((8, 128), jnp.float32)
    for e in range(E):
        past_end = (b_iota >= cum_end[e:e + 1, 0:1]).astype(jnp.float32)
        be = be + past_end
        sl = sl + jnp.where(nonempty[e:e + 1, 0:1], past_end, 0.0)
    be = jnp.minimum(be, last_e)
    sl = sl - 2.0 * jnp.floor(sl * 0.5)

    nxt_for = [None] * E
    nxt_for[E - 1] = jnp.zeros((1, 1), jnp.float32) + float(E - 1)
    for e in range(E - 2, -1, -1):
        cand = nxt_for[e + 1]
        cand = jnp.where(cand == float(e + 1), float(e), cand)
        nxt_for[e] = jnp.where(nonempty[e + 1:e + 2, 0:1], float(e + 1), cand)
    nxt = jnp.zeros((8, 128), jnp.float32)
    for e in range(E):
        in_region = ((b_iota >= blk_start[e:e + 1, 0:1])
                     & (b_iota < cum_end[e:e + 1, 0:1])).astype(jnp.float32)
        nxt = nxt + nxt_for[e] * in_region

    be_ref[...] = be.astype(jnp.int32)
    nxt_ref[...] = nxt.astype(jnp.int32)
    sl_ref[...] = sl.astype(jnp.int32)
    vld_ref[...] = (b_iota < cum_end[E - 1:E, 0:1]).astype(jnp.int32)


def _expert_ffn_kernel(be_ref, nxt_ref, sl_ref, vld_ref, x_ref, w1_hbm,
                       b1_ref, w2_hbm, b2_ref, y_ref, w1b, w2b, s1, s2):
    i = pl.program_id(0)
    e = be_ref[0, i]
    sl = sl_ref[0, i]
    nxt = nxt_ref[0, i]
    prev_e = be_ref[0, jnp.maximum(i - 1, 0)]
    boundary = jnp.logical_or(i == 0, e != prev_e)

    @pl.when(i == 0)
    def _():
        pltpu.make_async_copy(w1_hbm.at[e], w1b.at[sl], s1.at[sl]).start()
        pltpu.make_async_copy(w2_hbm.at[e], w2b.at[sl], s2.at[sl]).start()

    @pl.when(jnp.logical_and(boundary, nxt != e))
    def _():
        nsl = 1 - sl
        pltpu.make_async_copy(w1_hbm.at[nxt], w1b.at[nsl], s1.at[nsl]).start()
        pltpu.make_async_copy(w2_hbm.at[nxt], w2b.at[nsl], s2.at[nsl]).start()

    @pl.when(boundary)
    def _():
        pltpu.make_async_copy(w1_hbm.at[e], w1b.at[sl], s1.at[sl]).wait()
        pltpu.make_async_copy(w2_hbm.at[e], w2b.at[sl], s2.at[sl]).wait()

    @pl.when(vld_ref[0, i] > 0)
    def _():
        x_lo, x_hi = _unpack_halves(x_ref[...])
        w1c = w1b[sl]
        h = lax.dot_general(x_lo, w1c[:, :HW].astype(jnp.bfloat16),
                            (((1,), (1,)), ((), ())),
                            preferred_element_type=jnp.float32)
        h = h + lax.dot_general(x_hi, w1c[:, HW:].astype(jnp.bfloat16),
                                (((1,), (1,)), ((), ())),
                                preferred_element_type=jnp.float32)
        h = h + b1_ref[0]
        h = 0.5 * h * (1.0 + lax.erf(h * (1.0 / math.sqrt(2.0))))
        y = lax.dot_general(h.astype(jnp.bfloat16),
                            w2b[sl].astype(jnp.bfloat16),
                            (((1,), (1,)), ((), ())),
                            preferred_element_type=jnp.float32)
        y = y + b2_ref[0]
        ybf = y.astype(jnp.bfloat16)
        y_ref[...] = _pack_halves(ybf[:, :HW], ybf[:, HW:])


def _combine_ln_kernel(hid_ref, y0_ref, y1_ref, gates_ref, g_ref, b_ref, o_ref):
    i = pl.program_id(0)
    g = gates_ref[:, pl.ds(i * RB, RB)]
    rr = lax.broadcasted_iota(jnp.int32, (RB, RB), 0)
    cr = lax.broadcasted_iota(jnp.int32, (RB, RB), 1)
    ident = (rr == cr).astype(jnp.float32)
    gcols = lax.dot_general(ident, g, (((1,), (1,)), ((), ())),
                            preferred_element_type=jnp.float32)
    g0 = gcols[:, 0:1]
    g1 = gcols[:, 1:2]
    y0_lo, y0_hi = _unpack_halves(y0_ref[...])
    y1_lo, y1_hi = _unpack_halves(y1_ref[...])
    hid = hid_ref[...]
    c_lo = (hid[:, :HW] + g0 * y0_lo.astype(jnp.float32)
            + g1 * y1_lo.astype(jnp.float32))
    c_hi = (hid[:, HW:] + g0 * y0_hi.astype(jnp.float32)
            + g1 * y1_hi.astype(jnp.float32))
    s = (jnp.sum(c_lo, axis=1, keepdims=True)
         + jnp.sum(c_hi, axis=1, keepdims=True))
    mu = s * (1.0 / H)
    d_lo = c_lo - mu
    d_hi = c_hi - mu
    v = (jnp.sum(d_lo * d_lo, axis=1, keepdims=True)
         + jnp.sum(d_hi * d_hi, axis=1, keepdims=True))
    rs = lax.rsqrt(v * (1.0 / H) + EPS)
    o_ref[:, :HW] = d_lo * rs * g_ref[:, :HW] + b_ref[:, :HW]
    o_ref[:, HW:] = d_hi * rs * g_ref[:, HW:] + b_ref[:, HW:]


CHUNK = 64
NCH = (2 * N) // 32 // CHUNK


@functools.cache
def _sc_kernels():
    mesh = plsc.VectorSubcoreMesh(core_axis_name="c", subcore_axis_name="s")

    @functools.partial(
        pl.kernel, mesh=mesh,
        out_type=jax.ShapeDtypeStruct((MAXPAD, HW), jnp.int32),
        scratch_types=[
            pltpu.VMEM((2, 128), jnp.int32),
            pltpu.VMEM((128, HW), jnp.int32),
            pltpu.SemaphoreType.DMA,
            pltpu.SemaphoreType.DMA,
        ],
    )
    def sc_dispatch(x_hbm, pos_hbm, xs_hbm, idx_v, rows_v, s0, s1):
        wid = lax.axis_index("s") * 2 + lax.axis_index("c")
        pltpu.sync_copy(pos_hbm.at[wid], idx_v)
        pltpu.sync_copy(x_hbm.at[pl.ds(wid * 128, 128)], rows_v)
        c0 = pltpu.async_copy(rows_v, xs_hbm.at[idx_v.at[0]], s0)
        c1 = pltpu.async_copy(rows_v, xs_hbm.at[idx_v.at[1]], s1)
        c0.wait()
        c1.wait()

    @functools.partial(
        pl.kernel, mesh=mesh,
        out_type=jax.ShapeDtypeStruct((2 * N, HW), jnp.int32),
        scratch_types=[
            pltpu.VMEM((NCH, CHUNK), jnp.int32),
            pltpu.VMEM((2, CHUNK, HW), jnp.int32),
            pltpu.SemaphoreType.DMA,
            pltpu.SemaphoreType.DMA,
        ],
    )
    def sc_combine_gather(ys_hbm, pos_hbm, out_hbm, idx_v, rows_v, s0, s1):
        wid = lax.axis_index("s") * 2 + lax.axis_index("c")
        pltpu.sync_copy(pos_hbm.at[wid], idx_v)
        base = wid * 256
        sems = [s0, s1]
        copies = [None, None]
        copies[0] = pltpu.async_copy(ys_hbm.at[idx_v.at[0]], rows_v.at[0],
                                     sems[0])
        for j in range(NCH):
            b = j % 2
            if j + 1 < NCH:
                copies[1 - b] = pltpu.async_copy(
                    ys_hbm.at[idx_v.at[j + 1]], rows_v.at[1 - b],
                    sems[1 - b])
            copies[b].wait()
            pltpu.sync_copy(rows_v.at[b],
                            out_hbm.at[pl.ds(base + j * CHUNK, CHUNK)])

    return sc_dispatch, sc_combine_gather


def kernel(hidden_states, router_w, router_b, w1, b1, w2, b2, ln_g, ln_b):
    bsz, seqlen, hdim = hidden_states.shape
    x = hidden_states.reshape(N, H)

    (pos, gates, be_full, nxt_full, sl_full, vld_full, x_bf,
     pos_t) = pl.pallas_call(
        _router_plan_kernel,
        out_shape=[
            jax.ShapeDtypeStruct((2, N), jnp.int32),
            jax.ShapeDtypeStruct((2, N), jnp.float32),
            jax.ShapeDtypeStruct((8, 128), jnp.int32),
            jax.ShapeDtypeStruct((8, 128), jnp.int32),
            jax.ShapeDtypeStruct((8, 128), jnp.int32),
            jax.ShapeDtypeStruct((8, 128), jnp.int32),
            jax.ShapeDtypeStruct((N, HW), jnp.int32),
            jax.ShapeDtypeStruct((32, 2, 128), jnp.int32),
        ],
    )(x, router_w, router_b.reshape(E, 1))

    pos3 = pos.reshape(32, NCH, CHUNK)

    sc_dispatch, sc_combine_gather = _sc_kernels()
    x_sorted = sc_dispatch(x_bf, pos_t)

    grid_spec = pltpu.PrefetchScalarGridSpec(
        num_scalar_prefetch=4,
        grid=(MAX_BLOCKS,),
        in_specs=[
            pl.BlockSpec((BLK, HW), lambda i, be, nx, sl, v: (i, 0)),
            pl.BlockSpec(memory_space=pl.ANY),
            pl.BlockSpec((1, 1, F), lambda i, be, nx, sl, v: (be[0, i], 0, 0)),
            pl.BlockSpec(memory_space=pl.ANY),
            pl.BlockSpec((1, 1, H), lambda i, be, nx, sl, v: (be[0, i], 0, 0)),
        ],
        out_specs=pl.BlockSpec((BLK, HW), lambda i, be, nx, sl, v: (i, 0)),
        scratch_shapes=[
            pltpu.VMEM((2, F, H), jnp.float32),
            pltpu.VMEM((2, H, F), jnp.float32),
            pltpu.SemaphoreType.DMA((2,)),
            pltpu.SemaphoreType.DMA((2,)),
        ],
    )
    y_sorted = pl.pallas_call(
        _expert_ffn_kernel,
        grid_spec=grid_spec,
        out_shape=jax.ShapeDtypeStruct((MAXPAD, HW), jnp.int32),
    )(be_full, nxt_full, sl_full, vld_full,
      x_sorted, w1, b1.reshape(E, 1, F), w2, b2.reshape(E, 1, H))

    y_tok = sc_combine_gather(y_sorted, pos3)

    out = pl.pallas_call(
        _combine_ln_kernel,
        grid=(N // RB,),
        in_specs=[
            pl.BlockSpec((RB, H), lambda i: (i, 0)),
            pl.BlockSpec((RB, HW), lambda i: (i, 0)),
            pl.BlockSpec((RB, HW), lambda i: (i + N // RB, 0)),
            pl.BlockSpec((2, N), lambda i: (0, 0)),
            pl.BlockSpec((1, H), lambda i: (0, 0)),
            pl.BlockSpec((1, H), lambda i: (0, 0)),
        ],
        out_specs=pl.BlockSpec((RB, H), lambda i: (i, 0)),
        out_shape=jax.ShapeDtypeStruct((N, H), jnp.float32),
    )(x, y_tok, y_tok, gates, ln_g.reshape(1, H), ln_b.reshape(1, H))

    return out.reshape(bsz, seqlen, hdim)
